# baseline (speedup 1.0000x reference)
import functools

import jax
import jax.numpy as jnp
from jax import lax
from jax.experimental import pallas as pl
from jax.experimental.pallas import tpu as pltpu
from jax.experimental.pallas import tpu_sc as plsc

_NUM_EMB = 100000
_NUM_NEW = 1024
_TABLE = _NUM_EMB + _NUM_NEW
_FEATURES = 128
_BATCH = 16384
_HIST = 50

_NC, _NS = 2, 16
_NW = _NC * _NS
_B = _BATCH * _HIST
_IDXW = 128
_ROWS_PER_W = _B // _NW
_GROUPS = _ROWS_PER_W // _IDXW

_CP_BIG = 6256
_CP_BIG_LAST = _NUM_EMB - 15 * _CP_BIG
_CP_NEW = _NUM_NEW // _NS

_NBUF = 4
_GG = _GROUPS // _NBUF


def _gather_body(x_hbm, emb_hbm, new_hbm, out_hbm, table_s, idx_v, bufs,
                 csem, gsems, osems):
    c = lax.axis_index("c")
    s = lax.axis_index("s")
    wid = s * _NC + c
    irow0 = wid * _GROUPS
    out_base = wid * _ROWS_PER_W

    pltpu.sync_copy(x_hbm.at[pl.ds(irow0, _GROUPS)], idx_v)

    big0 = s * _CP_BIG

    @pl.when(s < _NS - 1)
    def _():
        pltpu.async_copy(emb_hbm.at[pl.ds(big0, _CP_BIG)],
                         table_s.at[c, pl.ds(big0, _CP_BIG)], csem)
        pltpu.make_async_copy(emb_hbm.at[pl.ds(big0, _CP_BIG)],
                              table_s.at[c, pl.ds(big0, _CP_BIG)], csem).wait()

    @pl.when(s == _NS - 1)
    def _():
        last0 = 15 * _CP_BIG
        pltpu.async_copy(emb_hbm.at[pl.ds(last0, _CP_BIG_LAST)],
                         table_s.at[c, pl.ds(last0, _CP_BIG_LAST)], csem)
        pltpu.make_async_copy(
            emb_hbm.at[pl.ds(last0, _CP_BIG_LAST)],
            table_s.at[c, pl.ds(last0, _CP_BIG_LAST)], csem).wait()

    new0 = s * _CP_NEW
    pltpu.sync_copy(new_hbm.at[pl.ds(new0, _CP_NEW)],
                    table_s.at[c, pl.ds(_NUM_EMB + new0, _CP_NEW)])
    plsc.subcore_barrier()

    def wait_gather(b):
        pltpu.make_async_copy(
            out_hbm.at[pl.ds(out_base, _IDXW)], bufs[b], gsems[b]).wait()

    def wait_scatter(b):
        pltpu.make_async_copy(
            bufs[b], out_hbm.at[pl.ds(out_base, _IDXW)], osems[b]).wait()

    for b in range(_NBUF):
        pltpu.async_copy(table_s.at[c].at[idx_v.at[b]], bufs[b], gsems[b])

    def body(gg, _):
        for b in range(_NBUF):
            i = gg * _NBUF + b
            wait_gather(b)
            pltpu.async_copy(
                bufs[b], out_hbm.at[pl.ds(out_base + i * _IDXW, _IDXW)],
                osems[b])

            @pl.when(gg < _GG - 1)
            def _():
                wait_scatter(b)
                pltpu.async_copy(
                    table_s.at[c].at[idx_v.at[i + _NBUF]], bufs[b], gsems[b])
        return 0

    lax.fori_loop(0, _GG, body, 0)
    for b in range(_NBUF):
        wait_scatter(b)


def kernel(x, embedding, new_embedding):
    x2d = x.reshape(-1).astype(jnp.int32).reshape(_B // _IDXW, _IDXW)

    mesh = plsc.VectorSubcoreMesh(core_axis_name="c", subcore_axis_name="s")
    run = pl.kernel(
        _gather_body,
        out_type=jax.ShapeDtypeStruct((_B, _FEATURES), jnp.float32),
        mesh=mesh,
        scratch_types=[
            pltpu.HBM((_NC, _TABLE, _FEATURES), jnp.float32),
            pltpu.VMEM((_GROUPS, _IDXW), jnp.int32),
            tuple(pltpu.VMEM((_IDXW, _FEATURES), jnp.float32)
                  for _ in range(_NBUF)),
            pltpu.SemaphoreType.DMA,
            tuple(pltpu.SemaphoreType.DMA for _ in range(_NBUF)),
            tuple(pltpu.SemaphoreType.DMA for _ in range(_NBUF)),
        ],
    )
    out = run(x2d, embedding, new_embedding)
    return out.reshape(_BATCH, _HIST, _FEATURES)


# speedup vs baseline: 3.5988x; 3.5988x over previous
import functools

import jax
import jax.numpy as jnp
from jax import lax
from jax.experimental import pallas as pl
from jax.experimental.pallas import tpu as pltpu
from jax.experimental.pallas import tpu_sc as plsc

_NUM_EMB = 100000
_NUM_NEW = 1024
_TABLE = _NUM_EMB + _NUM_NEW
_FEATURES = 128
_BATCH = 16384
_HIST = 50

_NC, _NS = 2, 16
_NW = _NC * _NS
_B = _BATCH * _HIST
_IDXW = 128
_ROWS_PER_W = _B // _NW
_GROUPS = _ROWS_PER_W // _IDXW

_CP_W = 128
_CP_FULL = _NUM_EMB // _CP_W
_CP_TAIL = _NUM_EMB - _CP_FULL * _CP_W
_CP_SLOTS = 49
_CP_NEW_CH = _NUM_NEW // _CP_W

_NBUF = 4
_GG = _GROUPS // _NBUF


def _gather_body(x_hbm, emb_hbm, new_hbm, out_hbm, table_s, idx_v, bufs,
                 gsems, osems):
    c = lax.axis_index("c")
    s = lax.axis_index("s")
    wid = s * _NC + c
    irow0 = wid * _GROUPS
    out_base = wid * _ROWS_PER_W

    pltpu.sync_copy(x_hbm.at[pl.ds(irow0, _GROUPS)], idx_v)

    def p0_wait_in(b):
        pltpu.make_async_copy(
            emb_hbm.at[pl.ds(0, _CP_W)], bufs[b], gsems[b]).wait()

    def p0_wait_out(b):
        pltpu.make_async_copy(
            bufs[b], table_s.at[c, pl.ds(0, _CP_W)], osems[b]).wait()

    def p0_fire_in(j, b):
        pltpu.async_copy(emb_hbm.at[pl.ds(j * _CP_W, _CP_W)], bufs[b],
                         gsems[b])

    def p0_fire_out(j, b):
        pltpu.async_copy(bufs[b], table_s.at[c, pl.ds(j * _CP_W, _CP_W)],
                         osems[b])

    for b in range(_NBUF):
        p0_fire_in(s + 16 * b, b)

    def p0_body(kk, _):
        for b in range(_NBUF):
            k = kk * _NBUF + b
            j = s + 16 * k
            p0_wait_in(b)
            p0_fire_out(j, b)
            knext = k + _NBUF

            @pl.when((knext < _CP_SLOTS - 1)
                     | ((knext == _CP_SLOTS - 1) & (s < 13)))
            def _():
                p0_wait_out(b)
                p0_fire_in(s + 16 * knext, b)
        return 0

    lax.fori_loop(0, (_CP_SLOTS - 1) // _NBUF, p0_body, 0)

    @pl.when(s < 13)
    def _():
        p0_wait_in(0)
        p0_fire_out(s + 16 * (_CP_SLOTS - 1), 0)
        p0_wait_out(0)

    @pl.when(s >= 13)
    def _():
        p0_wait_out(0)

    for b in range(1, _NBUF):
        p0_wait_out(b)

    @pl.when(s == 13)
    def _():
        pltpu.sync_copy(emb_hbm.at[pl.ds(_CP_FULL * _CP_W, _CP_TAIL)],
                        bufs[1].at[pl.ds(0, _CP_TAIL)])
        pltpu.sync_copy(bufs[1].at[pl.ds(0, _CP_TAIL)],
                        table_s.at[c, pl.ds(_CP_FULL * _CP_W, _CP_TAIL)])

    @pl.when(s < _CP_NEW_CH)
    def _():
        pltpu.sync_copy(new_hbm.at[pl.ds(s * _CP_W, _CP_W)], bufs[2])
        pltpu.sync_copy(bufs[2],
                        table_s.at[c, pl.ds(_NUM_EMB + s * _CP_W, _CP_W)])

    plsc.subcore_barrier()

    def wait_gather(b):
        pltpu.make_async_copy(
            out_hbm.at[pl.ds(out_base, _IDXW)], bufs[b], gsems[b]).wait()

    def wait_scatter(b):
        pltpu.make_async_copy(
            bufs[b], out_hbm.at[pl.ds(out_base, _IDXW)], osems[b]).wait()

    for b in range(_NBUF):
        pltpu.async_copy(table_s.at[c].at[idx_v.at[b]], bufs[b], gsems[b])

    def body(gg, _):
        for b in range(_NBUF):
            i = gg * _NBUF + b
            wait_gather(b)
            pltpu.async_copy(
                bufs[b], out_hbm.at[pl.ds(out_base + i * _IDXW, _IDXW)],
                osems[b])

            @pl.when(gg < _GG - 1)
            def _():
                wait_scatter(b)
                pltpu.async_copy(
                    table_s.at[c].at[idx_v.at[i + _NBUF]], bufs[b], gsems[b])
        return 0

    lax.fori_loop(0, _GG, body, 0)
    for b in range(_NBUF):
        wait_scatter(b)


def kernel(x, embedding, new_embedding):
    x2d = x.reshape(-1).astype(jnp.int32).reshape(_B // _IDXW, _IDXW)

    mesh = plsc.VectorSubcoreMesh(core_axis_name="c", subcore_axis_name="s")
    run = pl.kernel(
        _gather_body,
        out_type=jax.ShapeDtypeStruct((_B, _FEATURES), jnp.float32),
        mesh=mesh,
        scratch_types=[
            pltpu.HBM((_NC, _TABLE, _FEATURES), jnp.float32),
            pltpu.VMEM((_GROUPS, _IDXW), jnp.int32),
            tuple(pltpu.VMEM((_IDXW, _FEATURES), jnp.float32)
                  for _ in range(_NBUF)),
            tuple(pltpu.SemaphoreType.DMA for _ in range(_NBUF)),
            tuple(pltpu.SemaphoreType.DMA for _ in range(_NBUF)),
        ],
    )
    out = run(x2d, embedding, new_embedding)
    return out.reshape(_BATCH, _HIST, _FEATURES)


# speedup vs baseline: 6.1774x; 1.7165x over previous
import functools

import jax
import jax.numpy as jnp
from jax import lax
from jax.experimental import pallas as pl
from jax.experimental.pallas import tpu as pltpu
from jax.experimental.pallas import tpu_sc as plsc

_NUM_EMB = 100000
_NUM_NEW = 1024
_TABLE = _NUM_EMB + _NUM_NEW
_FEATURES = 128
_BATCH = 16384
_HIST = 50
_HPAD = 56

_NC, _NS = 2, 16
_NW = _NC * _NS
_BT_PER_W = _BATCH // _NW
_KB = 4
_GRP = _BT_PER_W // _KB
_ROWS = _KB * _HIST

_CP_CH = _NUM_EMB // _ROWS
_CP_SLOTS = 32
_NEW_CH = _NUM_NEW // _ROWS
_NEW_TAIL = _NUM_NEW - _NEW_CH * _ROWS

_NBUF = 2


def _gather_body(x_hbm, emb_hbm, new_hbm, out_hbm, table_s, idx_v, bufs,
                 gsems, osems):
    c = lax.axis_index("c")
    s = lax.axis_index("s")
    wid = s * _NC + c
    bt0 = wid * _BT_PER_W

    pltpu.sync_copy(x_hbm.at[pl.ds(bt0 * _HPAD, _BT_PER_W * _HPAD)], idx_v)

    def p0_wait_in(b):
        pltpu.make_async_copy(
            emb_hbm.at[pl.ds(0, _ROWS)], bufs[b], gsems[b]).wait()

    def p0_wait_out(b):
        pltpu.make_async_copy(
            bufs[b], table_s.at[c, pl.ds(0, _ROWS)], osems[b]).wait()

    def p0_fire_in(j, b):
        pltpu.async_copy(emb_hbm.at[pl.ds(j * _ROWS, _ROWS)], bufs[b],
                         gsems[b])

    def p0_fire_out(j, b):
        pltpu.async_copy(bufs[b], table_s.at[c, pl.ds(j * _ROWS, _ROWS)],
                         osems[b])

    for b in range(_NBUF):
        p0_fire_in(s + 16 * b, b)

    def p0_body(kk, _):
        for b in range(_NBUF):
            k = kk * _NBUF + b
            j = s + 16 * k
            act = (k < _CP_SLOTS - 1) | (s < 4)
            actn = (k + _NBUF < _CP_SLOTS - 1) | ((k + _NBUF == _CP_SLOTS - 1)
                                                  & (s < 4))

            @pl.when(act)
            def _():
                p0_wait_in(b)
                p0_fire_out(j, b)

            @pl.when(act & actn)
            def _():
                p0_wait_out(b)
                p0_fire_in(s + 16 * (k + _NBUF), b)
        return 0

    lax.fori_loop(0, _CP_SLOTS // _NBUF, p0_body, 0)
    for b in range(_NBUF):
        p0_wait_out(b)

    @pl.when(s < _NEW_CH)
    def _():
        pltpu.sync_copy(new_hbm.at[pl.ds(s * _ROWS, _ROWS)], bufs[0])
        pltpu.sync_copy(bufs[0],
                        table_s.at[c, pl.ds(_NUM_EMB + s * _ROWS, _ROWS)])

    @pl.when(s == _NEW_CH)
    def _():
        pltpu.sync_copy(new_hbm.at[pl.ds(_NEW_CH * _ROWS, _NEW_TAIL)],
                        bufs[0].at[pl.ds(0, _NEW_TAIL)])
        pltpu.sync_copy(
            bufs[0].at[pl.ds(0, _NEW_TAIL)],
            table_s.at[c, pl.ds(_NUM_EMB + _NEW_CH * _ROWS, _NEW_TAIL)])

    plsc.subcore_barrier()

    def wait_gather(b):
        pltpu.make_async_copy(
            table_s.at[c, pl.ds(0, _ROWS)], bufs[b], gsems[b]).wait()

    def wait_scatter(b):
        pltpu.make_async_copy(
            bufs[b], table_s.at[c, pl.ds(0, _ROWS)], osems[b]).wait()

    def fire_gather(g, b):
        for kb in range(_KB):
            i = g * _KB + kb
            pltpu.async_copy(
                table_s.at[c].at[idx_v.at[pl.ds(i * _HPAD, _HIST)]],
                bufs[b].at[pl.ds(kb * _HIST, _HIST)], gsems[b])

    def fire_scatter(g, b):
        for kb in range(_KB):
            pltpu.async_copy(bufs[b].at[pl.ds(kb * _HIST, _HIST)],
                             out_hbm.at[bt0 + g * _KB + kb], osems[b])

    for b in range(_NBUF):
        fire_gather(b, b)

    def body(gg, _):
        for b in range(_NBUF):
            g = gg * _NBUF + b
            wait_gather(b)
            fire_scatter(g, b)

            @pl.when(gg < _GRP // _NBUF - 1)
            def _():
                wait_scatter(b)
                fire_gather(g + _NBUF, b)
        return 0

    lax.fori_loop(0, _GRP // _NBUF, body, 0)
    for b in range(_NBUF):
        wait_scatter(b)


def kernel(x, embedding, new_embedding):
    xi = x.astype(jnp.int32)
    xpad = jnp.pad(xi, ((0, 0), (0, _HPAD - _HIST))).reshape(-1)

    mesh = plsc.VectorSubcoreMesh(core_axis_name="c", subcore_axis_name="s")
    run = pl.kernel(
        _gather_body,
        out_type=jax.ShapeDtypeStruct((_BATCH, _HIST, _FEATURES),
                                      jnp.float32),
        mesh=mesh,
        scratch_types=[
            pltpu.HBM((_NC, _TABLE, _FEATURES), jnp.float32),
            pltpu.VMEM((_BT_PER_W * _HPAD,), jnp.int32),
            tuple(pltpu.VMEM((_ROWS, _FEATURES), jnp.float32)
                  for _ in range(_NBUF)),
            tuple(pltpu.SemaphoreType.DMA for _ in range(_NBUF)),
            tuple(pltpu.SemaphoreType.DMA for _ in range(_NBUF)),
        ],
    )
    return run(xpad, embedding, new_embedding)
